# baseline (speedup 1.0000x reference)
import functools

import jax
import jax.numpy as jnp
from jax import lax
from jax.experimental import pallas as pl
from jax.experimental.pallas import tpu as pltpu
from jax.experimental.pallas import tpu_sc as plsc

_N_TOKENS = 32768
_D_MODEL = 4096
_NUM_EXPERTS = 64
_TOP_K = 8
_MM_BLOCK = 1024

_NC = 2
_NS = 16
_NW = _NC * _NS
_LANES = 16
_CHUNK = 512
_N_SPLITS = 2

_KEY_MIN = -(2 ** 31)


def _matmul_block(x_ref, w_ref, b_ref, out_ref):
    out_ref[...] = lax.dot_general(
        x_ref[...], w_ref[...], (((1,), (1,)), ((), ())),
        preferred_element_type=jnp.float32,
    ) + b_ref[...]


def _router_logits(x, W, b2d):
    grid = (x.shape[0] // _MM_BLOCK,)
    return pl.pallas_call(
        _matmul_block,
        grid=grid,
        in_specs=[
            pl.BlockSpec((_MM_BLOCK, _D_MODEL), lambda i: (i, 0)),
            pl.BlockSpec((_NUM_EXPERTS, _D_MODEL), lambda i: (0, 0)),
            pl.BlockSpec((1, _NUM_EXPERTS), lambda i: (0, 0)),
        ],
        out_specs=pl.BlockSpec((_MM_BLOCK, _NUM_EXPERTS), lambda i: (i, 0)),
        out_shape=jax.ShapeDtypeStruct((x.shape[0], _NUM_EXPERTS), jnp.float32),
        compiler_params=pltpu.CompilerParams(
            dimension_semantics=("parallel",),
        ),
    )(x, W, b2d)


def _topk_softmax_subblock(sb, lbuf, wbuf, ibuf):
    lanes = lax.iota(jnp.int32, _LANES)
    lrow0 = lanes * _NUM_EXPERTS + sb * (_LANES * _NUM_EXPERTS)

    def estep(e, t):
        v = plsc.load_gather(lbuf, [lrow0 + e])
        bits = plsc.bitcast(v, jnp.int32)
        key = bits ^ ((bits >> jnp.int32(31)) & jnp.int32(0x7FFFFFFF))
        key = (key & jnp.int32(~63)) | (jnp.int32(_NUM_EXPERTS - 1) - e)
        t = list(t)
        for j in range(_TOP_K):
            t[j], key = jnp.maximum(t[j], key), jnp.minimum(t[j], key)
        return tuple(t)

    init = tuple(jnp.full((_LANES,), _KEY_MIN, jnp.int32) for _ in range(_TOP_K))
    t = lax.fori_loop(0, _NUM_EXPERTS, estep, init, unroll=4)

    ti = [jnp.int32(_NUM_EXPERTS - 1) - (t[j] & jnp.int32(63))
          for j in range(_TOP_K)]
    tv = [plsc.load_gather(lbuf, [lrow0 + ti[j]]) for j in range(_TOP_K)]

    for a in range(1, _TOP_K):
        for bb in range(a, 0, -1):
            c = tv[bb - 1] < tv[bb]
            tv[bb - 1], tv[bb] = (jnp.where(c, tv[bb], tv[bb - 1]),
                                  jnp.where(c, tv[bb - 1], tv[bb]))
            ti[bb - 1], ti[bb] = (jnp.where(c, ti[bb], ti[bb - 1]),
                                  jnp.where(c, ti[bb - 1], ti[bb]))

    ew = [jnp.exp(tv[j] - tv[0]) for j in range(_TOP_K)]
    denom = ew[0]
    for j in range(1, _TOP_K):
        denom = denom + ew[j]
    inv = jnp.float32(1.0) / denom

    zero = jnp.zeros((_LANES,), jnp.float32)
    base = sb * (_LANES * _NUM_EXPERTS)
    for o in range(_LANES * _NUM_EXPERTS // _LANES):
        wbuf[pl.ds(base + o * _LANES, _LANES)] = zero
    ibase = lanes * _TOP_K + sb * (_LANES * _TOP_K)
    for j in range(_TOP_K):
        plsc.store_scatter(wbuf, [lrow0 + ti[j]], ew[j] * inv)
        plsc.store_scatter(ibuf, [ibase + j], ti[j])


def _gating_sc(logits_flat, n_tokens):
    rows_per_w = n_tokens // _NW
    mesh = plsc.VectorSubcoreMesh(core_axis_name="c", subcore_axis_name="s")

    @functools.partial(
        pl.kernel,
        out_type=[
            jax.ShapeDtypeStruct((n_tokens * _NUM_EXPERTS,), jnp.float32),
            jax.ShapeDtypeStruct((n_tokens * _TOP_K,), jnp.int32),
        ],
        mesh=mesh,
        scratch_types=[
            pltpu.VMEM((_CHUNK * _NUM_EXPERTS,), jnp.float32),
            pltpu.VMEM((_CHUNK * _NUM_EXPERTS,), jnp.float32),
            pltpu.VMEM((_CHUNK * _TOP_K,), jnp.int32),
        ],
        compiler_params=pltpu.CompilerParams(needs_layout_passes=False),
    )
    def body(logits_hbm, w_hbm, i_hbm, lbuf, wbuf, ibuf):
        wid = lax.axis_index("s") * _NC + lax.axis_index("c")
        row0 = wid * rows_per_w

        def chunk(ci, carry):
            r0 = row0 + ci * _CHUNK
            pltpu.sync_copy(
                logits_hbm.at[pl.ds(r0 * _NUM_EXPERTS, _CHUNK * _NUM_EXPERTS)],
                lbuf)

            def sub(sb, c2):
                _topk_softmax_subblock(sb, lbuf, wbuf, ibuf)
                return c2

            lax.fori_loop(0, _CHUNK // _LANES, sub, 0)
            pltpu.sync_copy(
                wbuf,
                w_hbm.at[pl.ds(r0 * _NUM_EXPERTS, _CHUNK * _NUM_EXPERTS)])
            pltpu.sync_copy(
                ibuf, i_hbm.at[pl.ds(r0 * _TOP_K, _CHUNK * _TOP_K)])
            return carry

        lax.fori_loop(0, rows_per_w // _CHUNK, chunk, 0)

    return body(logits_flat)


@jax.jit
def kernel(x, W, b):
    b2d = b.reshape(1, _NUM_EXPERTS)
    split = _N_TOKENS // _N_SPLITS
    w_parts, i_parts = [], []
    for c in range(_N_SPLITS):
        logits = _router_logits(
            lax.slice_in_dim(x, c * split, (c + 1) * split, axis=0), W, b2d)
        w_flat, i_flat = _gating_sc(logits.reshape(-1), split)
        w_parts.append(w_flat.reshape(split, _NUM_EXPERTS))
        i_parts.append(i_flat.reshape(split, _TOP_K))
    if _N_SPLITS == 1:
        return w_parts[0], i_parts[0]
    return (jnp.concatenate(w_parts, axis=0),
            jnp.concatenate(i_parts, axis=0))


# speedup vs baseline: 2.4676x; 2.4676x over previous
import functools

import jax
import jax.numpy as jnp
from jax import lax
from jax.experimental import pallas as pl
from jax.experimental.pallas import tpu as pltpu
from jax.experimental.pallas import tpu_sc as plsc

_N_TOKENS = 32768
_D_MODEL = 4096
_NUM_EXPERTS = 64
_TOP_K = 8
_MM_BLOCK = 1024

_NC = 2
_NS = 16
_NW = _NC * _NS
_LANES = 16
_CHUNK = 512
_ROWS_PER_W = _N_TOKENS // _NW

_KEY_MIN = -(2 ** 31)


def _matmul_block_t(x_ref, w_ref, b_ref, out_ref):
    out_ref[...] = lax.dot_general(
        w_ref[...], x_ref[...], (((1,), (1,)), ((), ())),
        preferred_element_type=jnp.float32,
    ) + b_ref[...]


def _router_logits_t(x, W, b_bcast):
    grid = (x.shape[0] // _MM_BLOCK,)
    return pl.pallas_call(
        _matmul_block_t,
        grid=grid,
        in_specs=[
            pl.BlockSpec((_MM_BLOCK, _D_MODEL), lambda i: (i, 0)),
            pl.BlockSpec((_NUM_EXPERTS, _D_MODEL), lambda i: (0, 0)),
            pl.BlockSpec((_NUM_EXPERTS, _MM_BLOCK), lambda i: (0, 0)),
        ],
        out_specs=pl.BlockSpec((_NUM_EXPERTS, _MM_BLOCK), lambda i: (0, i)),
        out_shape=jax.ShapeDtypeStruct((_NUM_EXPERTS, x.shape[0]), jnp.float32),
        compiler_params=pltpu.CompilerParams(
            dimension_semantics=("arbitrary",),
        ),
    )(x, W, b_bcast)


def _topk_softmax_subblock(sb, lbuf, wbuf, ibuf):
    lanes = lax.iota(jnp.int32, _LANES)
    col0 = sb * _LANES
    lrow0 = lanes * _NUM_EXPERTS + sb * (_LANES * _NUM_EXPERTS)

    t = [jnp.full((_LANES,), _KEY_MIN, jnp.int32) for _ in range(_TOP_K)]
    for e in range(_NUM_EXPERTS):
        v = lbuf[e, pl.ds(col0, _LANES)]
        bits = plsc.bitcast(v, jnp.int32)
        key = bits ^ ((bits >> jnp.int32(31)) & jnp.int32(0x7FFFFFFF))
        key = (key & jnp.int32(~63)) | jnp.int32(_NUM_EXPERTS - 1 - e)
        for j in range(_TOP_K):
            t[j], key = jnp.maximum(t[j], key), jnp.minimum(t[j], key)

    ti = [jnp.int32(_NUM_EXPERTS - 1) - (t[j] & jnp.int32(63))
          for j in range(_TOP_K)]
    cols = col0 + lanes
    tv = [plsc.load_gather(lbuf, [ti[j], cols]) for j in range(_TOP_K)]

    for a in range(1, _TOP_K):
        for bb in range(a, 0, -1):
            c = tv[bb - 1] < tv[bb]
            tv[bb - 1], tv[bb] = (jnp.where(c, tv[bb], tv[bb - 1]),
                                  jnp.where(c, tv[bb - 1], tv[bb]))
            ti[bb - 1], ti[bb] = (jnp.where(c, ti[bb], ti[bb - 1]),
                                  jnp.where(c, ti[bb - 1], ti[bb]))

    ew = [jnp.exp(tv[j] - tv[0]) for j in range(_TOP_K)]
    denom = ew[0]
    for j in range(1, _TOP_K):
        denom = denom + ew[j]
    inv = jnp.float32(1.0) / denom

    zero = jnp.zeros((_LANES,), jnp.float32)
    base = sb * (_LANES * _NUM_EXPERTS)
    for o in range(_LANES * _NUM_EXPERTS // _LANES):
        wbuf[pl.ds(base + o * _LANES, _LANES)] = zero
    ibase = lanes * _TOP_K + sb * (_LANES * _TOP_K)
    for j in range(_TOP_K):
        plsc.store_scatter(wbuf, [lrow0 + ti[j]], ew[j] * inv)
        plsc.store_scatter(ibuf, [ibase + j], ti[j])


def _gating_sc(logits_t):
    mesh = plsc.VectorSubcoreMesh(core_axis_name="c", subcore_axis_name="s")

    @functools.partial(
        pl.kernel,
        out_type=[
            jax.ShapeDtypeStruct((_N_TOKENS * _NUM_EXPERTS,), jnp.float32),
            jax.ShapeDtypeStruct((_N_TOKENS * _TOP_K,), jnp.int32),
        ],
        mesh=mesh,
        scratch_types=[
            pltpu.VMEM((_NUM_EXPERTS, _CHUNK), jnp.float32),
            pltpu.VMEM((_CHUNK * _NUM_EXPERTS,), jnp.float32),
            pltpu.VMEM((_CHUNK * _TOP_K,), jnp.int32),
        ],
        compiler_params=pltpu.CompilerParams(needs_layout_passes=False),
    )
    def body(logits_hbm, w_hbm, i_hbm, lbuf, wbuf, ibuf):
        wid = lax.axis_index("s") * _NC + lax.axis_index("c")
        row0 = wid * _ROWS_PER_W

        def chunk(ci, carry):
            r0 = row0 + ci * _CHUNK
            pltpu.sync_copy(logits_hbm.at[:, pl.ds(r0, _CHUNK)], lbuf)

            def sub(sb, c2):
                _topk_softmax_subblock(sb, lbuf, wbuf, ibuf)
                return c2

            lax.fori_loop(0, _CHUNK // _LANES, sub, 0)
            pltpu.sync_copy(
                wbuf,
                w_hbm.at[pl.ds(r0 * _NUM_EXPERTS, _CHUNK * _NUM_EXPERTS)])
            pltpu.sync_copy(
                ibuf, i_hbm.at[pl.ds(r0 * _TOP_K, _CHUNK * _TOP_K)])
            return carry

        lax.fori_loop(0, _ROWS_PER_W // _CHUNK, chunk, 0)

    return body(logits_t)


@jax.jit
def kernel(x, W, b):
    b_bcast = jnp.broadcast_to(b.reshape(_NUM_EXPERTS, 1),
                               (_NUM_EXPERTS, _MM_BLOCK))
    logits_t = _router_logits_t(x, W, b_bcast)
    w_flat, i_flat = _gating_sc(logits_t)
    return (w_flat.reshape(_N_TOKENS, _NUM_EXPERTS),
            i_flat.reshape(_N_TOKENS, _TOP_K))


# speedup vs baseline: 2.4718x; 1.0017x over previous
import functools

import jax
import jax.numpy as jnp
from jax import lax
from jax.experimental import pallas as pl
from jax.experimental.pallas import tpu as pltpu
from jax.experimental.pallas import tpu_sc as plsc

_N_TOKENS = 32768
_D_MODEL = 4096
_NUM_EXPERTS = 64
_TOP_K = 8
_MM_BLOCK = 1024

_NC = 2
_NS = 16
_NW = _NC * _NS
_LANES = 16
_CHUNK = 512
_ROWS_PER_W = _N_TOKENS // _NW

_KEY_MIN = -(2 ** 31)


def _matmul_block_t(x_ref, w_ref, b_ref, out_ref):
    lg = lax.dot_general(
        w_ref[...], x_ref[...], (((1,), (1,)), ((), ())),
        preferred_element_type=jnp.float32,
    ) + b_ref[...]
    out_ref[...] = lg.reshape(_NUM_EXPERTS, _MM_BLOCK // 128, 128)


def _router_logits_t(x, W, b_bcast):
    grid = (x.shape[0] // _MM_BLOCK,)
    return pl.pallas_call(
        _matmul_block_t,
        grid=grid,
        in_specs=[
            pl.BlockSpec((_MM_BLOCK, _D_MODEL), lambda i: (i, 0)),
            pl.BlockSpec((_NUM_EXPERTS, _D_MODEL), lambda i: (0, 0)),
            pl.BlockSpec((_NUM_EXPERTS, _MM_BLOCK), lambda i: (0, 0)),
        ],
        out_specs=pl.BlockSpec(
            (_NUM_EXPERTS, _MM_BLOCK // 128, 128), lambda i: (0, i, 0)),
        out_shape=jax.ShapeDtypeStruct(
            (_NUM_EXPERTS, x.shape[0] // 128, 128), jnp.float32),
        compiler_params=pltpu.CompilerParams(
            dimension_semantics=("arbitrary",),
        ),
    )(x, W, b_bcast)


def _topk_softmax_subblock(sb, half, lbuf, wbuf, ibuf):
    lanes = lax.iota(jnp.int32, _LANES)
    col0 = half * _CHUNK + sb * _LANES
    cmid = col0 // 128
    clo = col0 % 128
    lrow0 = lanes * _NUM_EXPERTS + sb * (_LANES * _NUM_EXPERTS)

    t = [jnp.full((_LANES,), _KEY_MIN, jnp.int32) for _ in range(_TOP_K)]
    for e in range(_NUM_EXPERTS):
        v = lbuf[e, cmid, pl.ds(clo, _LANES)]
        bits = plsc.bitcast(v, jnp.int32)
        key = bits ^ ((bits >> jnp.int32(31)) & jnp.int32(0x7FFFFFFF))
        key = (key & jnp.int32(~63)) | jnp.int32(_NUM_EXPERTS - 1 - e)
        for j in range(_TOP_K):
            t[j], key = jnp.maximum(t[j], key), jnp.minimum(t[j], key)

    ti = [jnp.int32(_NUM_EXPERTS - 1) - (t[j] & jnp.int32(63))
          for j in range(_TOP_K)]
    cmids = jnp.full((_LANES,), cmid, jnp.int32)
    clos = clo + lanes
    tv = [plsc.load_gather(lbuf, [ti[j], cmids, clos]) for j in range(_TOP_K)]

    for a in range(1, _TOP_K):
        for bb in range(a, 0, -1):
            c = tv[bb - 1] < tv[bb]
            tv[bb - 1], tv[bb] = (jnp.where(c, tv[bb], tv[bb - 1]),
                                  jnp.where(c, tv[bb - 1], tv[bb]))
            ti[bb - 1], ti[bb] = (jnp.where(c, ti[bb], ti[bb - 1]),
                                  jnp.where(c, ti[bb - 1], ti[bb]))

    ew = [jnp.exp(tv[j] - tv[0]) for j in range(_TOP_K)]
    denom = ew[0]
    for j in range(1, _TOP_K):
        denom = denom + ew[j]
    inv = jnp.float32(1.0) / denom

    zero = jnp.zeros((_LANES,), jnp.float32)
    base = sb * (_LANES * _NUM_EXPERTS)
    for o in range(_LANES * _NUM_EXPERTS // _LANES):
        wbuf[pl.ds(base + o * _LANES, _LANES)] = zero
    ibase = lanes * _TOP_K + sb * (_LANES * _TOP_K)
    for j in range(_TOP_K):
        plsc.store_scatter(wbuf, [lrow0 + ti[j]], ew[j] * inv)
        plsc.store_scatter(ibuf, [ibase + j], ti[j])


def _gating_sc(logits_t):
    mesh = plsc.VectorSubcoreMesh(core_axis_name="c", subcore_axis_name="s")

    @functools.partial(
        pl.kernel,
        out_type=[
            jax.ShapeDtypeStruct((_N_TOKENS * _NUM_EXPERTS,), jnp.float32),
            jax.ShapeDtypeStruct((_N_TOKENS * _TOP_K,), jnp.int32),
        ],
        mesh=mesh,
        scratch_types=[
            pltpu.VMEM((_NUM_EXPERTS, _ROWS_PER_W // 128, 128), jnp.float32),
            pltpu.VMEM((_CHUNK * _NUM_EXPERTS,), jnp.float32),
            pltpu.VMEM((_CHUNK * _TOP_K,), jnp.int32),
        ],
        compiler_params=pltpu.CompilerParams(needs_layout_passes=False),
    )
    def body(logits_hbm, w_hbm, i_hbm, lbuf, wbuf, ibuf):
        wid = lax.axis_index("s") * _NC + lax.axis_index("c")
        row0 = wid * _ROWS_PER_W
        pltpu.sync_copy(
            logits_hbm.at[:, pl.ds(pl.multiple_of(row0 // 128, 8),
                                   _ROWS_PER_W // 128)], lbuf)
        for half in range(_ROWS_PER_W // _CHUNK):
            def sub(sb, c2, half=half):
                _topk_softmax_subblock(sb, half, lbuf, wbuf, ibuf)
                return c2

            lax.fori_loop(0, _CHUNK // _LANES, sub, 0)
            r0 = row0 + half * _CHUNK
            pltpu.sync_copy(
                wbuf,
                w_hbm.at[pl.ds(r0 * _NUM_EXPERTS, _CHUNK * _NUM_EXPERTS)])
            pltpu.sync_copy(
                ibuf, i_hbm.at[pl.ds(r0 * _TOP_K, _CHUNK * _TOP_K)])

    return body(logits_t)


@jax.jit
def kernel(x, W, b):
    b_bcast = jnp.broadcast_to(b.reshape(_NUM_EXPERTS, 1),
                               (_NUM_EXPERTS, _MM_BLOCK))
    logits_t = _router_logits_t(x, W, b_bcast)
    w_flat, i_flat = _gating_sc(logits_t)
    return (w_flat.reshape(_N_TOKENS, _NUM_EXPERTS),
            i_flat.reshape(_N_TOKENS, _TOP_K))
